# baseline (speedup 1.0000x reference)
import functools

import jax
import jax.numpy as jnp
from jax import lax
from jax.experimental import pallas as pl
from jax.experimental.pallas import tpu as pltpu
from jax.experimental.pallas import tpu_sc as plsc

N = 10000
E = 320000
D_IN = 128
HID = 3
D_T = 128

NC = 2
NS = 16
NW = NC * NS

NPAD = 10240
EPW = E // NW
PW = 8
RPS = NPAD // NS


def _make_edge_pass_body(gsel, ssel):
    def body(ei_h, table_h, zeros_h, out_h, gbuf, sbuf, vbuf, acc, sem):
        c = lax.axis_index("c")
        s = lax.axis_index("s")
        wid = c * NS + s
        off = pl.multiple_of(wid * EPW, 8)
        pltpu.sync_copy(ei_h.at[gsel, pl.ds(off, EPW)], gbuf)
        pltpu.sync_copy(ei_h.at[ssel, pl.ds(off, EPW)], sbuf)
        gather = pltpu.async_copy(table_h.at[gbuf], vbuf, sem)
        pltpu.sync_copy(zeros_h.at[pl.ds(s * RPS, RPS)],
                        acc.at[pl.ds(s * RPS, RPS)])
        gather.wait()
        plsc.subcore_barrier()
        pltpu.sync_copy(vbuf, acc.at[sbuf], add=True)
        plsc.subcore_barrier()
        pltpu.sync_copy(acc.at[pl.ds(s * RPS, RPS)],
                        out_h.at[c, pl.ds(s * RPS, RPS)])
    return body


@functools.cache
def _edge_pass_kernel(gsel, ssel):
    return pl.kernel(
        _make_edge_pass_body(gsel, ssel),
        out_type=jax.ShapeDtypeStruct((NC, NPAD, PW), jnp.float32),
        mesh=plsc.VectorSubcoreMesh(
            core_axis_name="c", subcore_axis_name="s",
            num_cores=NC, num_subcores=NS),
        scratch_types=[
            pltpu.VMEM((EPW,), jnp.int32),
            pltpu.VMEM((EPW,), jnp.int32),
            pltpu.VMEM((EPW, PW), jnp.float32),
            pltpu.VMEM_SHARED((NPAD, PW), jnp.float32),
            pltpu.SemaphoreType.DMA,
        ],
        compiler_params=pltpu.CompilerParams(use_tc_tiling_on_sc=False),
    )


def _edge_pass(ei, gsel, ssel, table, zeros):
    return _edge_pass_kernel(gsel, ssel)(ei, table, zeros)


BN = 1000
GRID = N // BN


def _tc1_body(x_ref, w10_ref, w11_ref, b1_ref, degp_ref,
              z1_ref, xw10_ref, dinv_ref):
    dp = degp_ref[...]
    deg = dp[0, :, :1] + dp[1, :, :1]
    dinv = jnp.where(deg > 0, lax.rsqrt(deg), 0.0)
    x = x_ref[...]
    y1 = jnp.dot(x, w11_ref[...], preferred_element_type=jnp.float32)
    y0 = jnp.dot(x, w10_ref[...], preferred_element_type=jnp.float32)
    z = dinv * y1
    z1_ref[...] = jnp.concatenate(
        [z, jnp.zeros((BN, PW - HID), jnp.float32)], axis=1)
    xw10_ref[...] = y0 + b1_ref[...]
    dinv_ref[...] = dinv


def _tc2_body(t1p_ref, dinv_ref, xw10_ref,
              z2_ref, h_ref):
    tp = t1p_ref[...]
    dinv = dinv_ref[...]
    t1 = -dinv * (tp[0, :, :HID] + tp[1, :, :HID])
    h = jnp.maximum(xw10_ref[...] + t1, 0.0)
    z2_ref[...] = jnp.concatenate(
        [dinv * h, jnp.zeros((BN, PW - HID), jnp.float32)], axis=1)
    h_ref[...] = h


def _tc3_body(t2p_ref, dinv_ref, h_ref, w20_ref, b2_ref, w21_ref,
              wl_ref, bl_ref, out_ref):
    tp = t2p_ref[...]
    t2 = -dinv_ref[...] * (tp[0, :, :HID] + tp[1, :, :HID])
    h2 = jnp.maximum(
        jnp.dot(h_ref[...], w20_ref[...], preferred_element_type=jnp.float32)
        + jnp.dot(t2, w21_ref[...], preferred_element_type=jnp.float32)
        + b2_ref[...],
        0.0)
    out_ref[...] = (
        jnp.dot(h2, wl_ref[...], preferred_element_type=jnp.float32)
        + bl_ref[...])


def _row_block(shape2):
    return pl.BlockSpec(shape2, lambda i: (i, 0))


def _full(shape):
    ndim = len(shape)
    return pl.BlockSpec(shape, lambda i: (0,) * ndim)


def _partials_block():
    return pl.BlockSpec((NC, BN, PW), lambda i: (0, i, 0))


_tc1 = pl.pallas_call(
    _tc1_body,
    grid=(GRID,),
    in_specs=[
        _row_block((BN, D_IN)),
        _full((D_IN, HID)),
        _full((D_IN, HID)),
        _full((1, HID)),
        _partials_block(),
    ],
    out_specs=(
        _row_block((BN, PW)),
        _row_block((BN, HID)),
        _row_block((BN, 1)),
    ),
    out_shape=(
        jax.ShapeDtypeStruct((N, PW), jnp.float32),
        jax.ShapeDtypeStruct((N, HID), jnp.float32),
        jax.ShapeDtypeStruct((N, 1), jnp.float32),
    ),
)

_tc2 = pl.pallas_call(
    _tc2_body,
    grid=(GRID,),
    in_specs=[
        _partials_block(),
        _row_block((BN, 1)),
        _row_block((BN, HID)),
    ],
    out_specs=(
        _row_block((BN, PW)),
        _row_block((BN, HID)),
    ),
    out_shape=(
        jax.ShapeDtypeStruct((N, PW), jnp.float32),
        jax.ShapeDtypeStruct((N, HID), jnp.float32),
    ),
)

_tc3 = pl.pallas_call(
    _tc3_body,
    grid=(GRID,),
    in_specs=[
        _partials_block(),
        _row_block((BN, 1)),
        _row_block((BN, HID)),
        _full((HID, D_T)),
        _full((1, D_T)),
        _full((HID, D_T)),
        _full((D_T, D_T)),
        _full((1, D_T)),
    ],
    out_specs=_row_block((BN, D_T)),
    out_shape=jax.ShapeDtypeStruct((N, D_T), jnp.float32),
)


def kernel(x, edge_index, W1_0, W1_1, b1, W2_0, W2_1, b2, Wl, bl):
    ones_t = jnp.ones((N, PW), jnp.float32)
    zeros_t = jnp.zeros((NPAD, PW), jnp.float32)

    degp = _edge_pass(edge_index, 0, 0, ones_t, zeros_t)
    z1, xw10, dinv = _tc1(x, W1_0, W1_1, b1.reshape(1, HID), degp)
    t1p = _edge_pass(edge_index, 0, 1, z1, zeros_t)
    z2, h = _tc2(t1p, dinv, xw10)
    t2p = _edge_pass(edge_index, 0, 1, z2, zeros_t)
    outp = _tc3(t2p, dinv, h, W2_0, b2.reshape(1, D_T), W2_1,
                Wl, bl.reshape(1, D_T))
    return outp


# speedup vs baseline: 1.0384x; 1.0384x over previous
import functools

import jax
import jax.numpy as jnp
from jax import lax
from jax.experimental import pallas as pl
from jax.experimental.pallas import tpu as pltpu
from jax.experimental.pallas import tpu_sc as plsc

N = 10000
E = 320000
D_IN = 128
HID = 3
D_T = 128

NC = 2
NS = 16
NW = NC * NS

NPAD = 10240
EPW = E // NW
PW = 8
RPS = NPAD // NS


def _make_edge_pass_body(gsel, ssel):
    def body(ei_h, table_h, zeros_h, out_h, gbuf, sbuf, vbuf, acc, sem):
        c = lax.axis_index("c")
        s = lax.axis_index("s")
        wid = c * NS + s
        off = pl.multiple_of(wid * EPW, 8)
        pltpu.sync_copy(ei_h.at[gsel, pl.ds(off, EPW)], gbuf)
        pltpu.sync_copy(ei_h.at[ssel, pl.ds(off, EPW)], sbuf)
        gather = pltpu.async_copy(table_h.at[gbuf], vbuf, sem)
        pltpu.sync_copy(zeros_h.at[pl.ds(s * RPS, RPS)],
                        acc.at[pl.ds(s * RPS, RPS)])
        gather.wait()
        plsc.subcore_barrier()
        pltpu.sync_copy(vbuf, acc.at[sbuf], add=True)
        plsc.subcore_barrier()
        pltpu.sync_copy(acc.at[pl.ds(s * RPS, RPS)],
                        out_h.at[c, pl.ds(s * RPS, RPS)])
    return body


@functools.cache
def _edge_pass_kernel(gsel, ssel):
    return pl.kernel(
        _make_edge_pass_body(gsel, ssel),
        out_type=jax.ShapeDtypeStruct((NC, NPAD, PW), jnp.float32),
        mesh=plsc.VectorSubcoreMesh(
            core_axis_name="c", subcore_axis_name="s",
            num_cores=NC, num_subcores=NS),
        scratch_types=[
            pltpu.VMEM((EPW,), jnp.int32),
            pltpu.VMEM((EPW,), jnp.int32),
            pltpu.VMEM((EPW, PW), jnp.float32),
            pltpu.VMEM_SHARED((NPAD, PW), jnp.float32),
            pltpu.SemaphoreType.DMA,
        ],
        compiler_params=pltpu.CompilerParams(use_tc_tiling_on_sc=False),
    )


def _edge_pass(ei, gsel, ssel, table, zeros):
    return _edge_pass_kernel(gsel, ssel)(ei, table, zeros)


def _tc1_body(x_ref, w10_ref, w11_ref, b1_ref, degp_ref,
              z1_ref, xw10_ref, dinv_ref):
    dp = degp_ref[...]
    deg = dp[0, :N, :1] + dp[1, :N, :1]
    dinv = jnp.where(deg > 0, lax.rsqrt(deg), 0.0)
    x = x_ref[...]
    y1 = jnp.dot(x, w11_ref[...], preferred_element_type=jnp.float32)
    y0 = jnp.dot(x, w10_ref[...], preferred_element_type=jnp.float32)
    z = dinv * y1
    z1_ref[...] = jnp.concatenate(
        [z, jnp.zeros((N, PW - HID), jnp.float32)], axis=1)
    xw10_ref[...] = y0 + b1_ref[...]
    dinv_ref[...] = dinv


def _tc2_body(t1p_ref, dinv_ref, xw10_ref,
              z2_ref, h_ref):
    tp = t1p_ref[...]
    dinv = dinv_ref[...]
    t1 = -dinv * (tp[0, :N, :HID] + tp[1, :N, :HID])
    h = jnp.maximum(xw10_ref[...] + t1, 0.0)
    z2_ref[...] = jnp.concatenate(
        [dinv * h, jnp.zeros((N, PW - HID), jnp.float32)], axis=1)
    h_ref[...] = h


def _tc3_body(t2p_ref, dinv_ref, h_ref, w20_ref, b2_ref, w21_ref,
              wl_ref, bl_ref, out_ref):
    tp = t2p_ref[...]
    t2 = -dinv_ref[...] * (tp[0, :N, :HID] + tp[1, :N, :HID])
    h2 = jnp.maximum(
        jnp.dot(h_ref[...], w20_ref[...], preferred_element_type=jnp.float32)
        + jnp.dot(t2, w21_ref[...], preferred_element_type=jnp.float32)
        + b2_ref[...],
        0.0)
    out_ref[...] = (
        jnp.dot(h2, wl_ref[...], preferred_element_type=jnp.float32)
        + bl_ref[...])


_tc1 = pl.pallas_call(
    _tc1_body,
    out_shape=(
        jax.ShapeDtypeStruct((N, PW), jnp.float32),
        jax.ShapeDtypeStruct((N, HID), jnp.float32),
        jax.ShapeDtypeStruct((N, 1), jnp.float32),
    ),
)

_tc2 = pl.pallas_call(
    _tc2_body,
    out_shape=(
        jax.ShapeDtypeStruct((N, PW), jnp.float32),
        jax.ShapeDtypeStruct((N, HID), jnp.float32),
    ),
)

_tc3 = pl.pallas_call(
    _tc3_body,
    out_shape=jax.ShapeDtypeStruct((N, D_T), jnp.float32),
)


def kernel(x, edge_index, W1_0, W1_1, b1, W2_0, W2_1, b2, Wl, bl):
    ones_t = jnp.ones((N, PW), jnp.float32)
    zeros_t = jnp.zeros((NPAD, PW), jnp.float32)

    degp = _edge_pass(edge_index, 0, 0, ones_t, zeros_t)
    z1, xw10, dinv = _tc1(x, W1_0, W1_1, b1.reshape(1, HID), degp)
    t1p = _edge_pass(edge_index, 0, 1, z1, zeros_t)
    z2, h = _tc2(t1p, dinv, xw10)
    t2p = _edge_pass(edge_index, 0, 1, z2, zeros_t)
    outp = _tc3(t2p, dinv, h, W2_0, b2.reshape(1, D_T), W2_1,
                Wl, bl.reshape(1, D_T))
    return outp
